# Initial kernel scaffold; baseline (speedup 1.0000x reference)
import jax
import jax.numpy as jnp
from jax.experimental import pallas as pl


def kernel(x, table):
    raise NotImplementedError("write your pallas kernel here")


# speedup vs baseline: 1.5735x; 1.5735x over previous
import functools
import math

import jax
import jax.numpy as jnp
from jax import lax
from jax.experimental import pallas as pl
from jax.experimental.pallas import tpu as pltpu
from jax.experimental.pallas import tpu_sc as plsc

D_DIM = 512
VOCAB = 256
SCALE = math.sqrt(float(D_DIM))


def _make_sc_kernel(B: int):
    info = plsc.get_sparse_core_info()
    NC, NS, L = info.num_cores, info.num_subcores, info.num_lanes
    NW = NC * NS
    assert B % NW == 0
    b_per_w = B // NW
    CH = 128
    assert b_per_w % CH == 0
    n_ch = b_per_w // CH

    mesh = plsc.VectorSubcoreMesh(core_axis_name="c", subcore_axis_name="s")

    @functools.partial(
        pl.kernel,
        mesh=mesh,
        out_type=jax.ShapeDtypeStruct((B, D_DIM), jnp.float32),
        scratch_types=[
            pltpu.VMEM((b_per_w,), jnp.int32),
            pltpu.VMEM((CH, D_DIM), jnp.float32),
            pltpu.SemaphoreType.DMA,
        ],
    )
    def emb_kernel(x_hbm, table_hbm, out_hbm, idx_v, buf, sem):
        cid = lax.axis_index("c")
        sid = lax.axis_index("s")
        wid = sid * NC + cid

        base = wid * b_per_w
        pltpu.sync_copy(x_hbm.at[pl.ds(base, b_per_w)], idx_v)

        def chunk_body(c, carry):
            off = c * CH
            pltpu.async_copy(
                table_hbm.at[idx_v.at[pl.ds(off, CH)]], buf, sem
            ).wait()

            def row_body(r, rcarry):
                for j in range(D_DIM // L):
                    buf[r, pl.ds(j * L, L)] = buf[r, pl.ds(j * L, L)] * SCALE
                return rcarry

            lax.fori_loop(0, CH, row_body, 0)
            pltpu.sync_copy(buf, out_hbm.at[pl.ds(base + off, CH)])
            return carry

        lax.fori_loop(0, n_ch, chunk_body, 0)

    return emb_kernel


def kernel(x, table):
    B = x.shape[0] * x.shape[1]
    out = _make_sc_kernel(B)(x.reshape(B), table)
    return out.reshape(x.shape + (D_DIM,))


# speedup vs baseline: 1.8515x; 1.1766x over previous
import functools
import math

import jax
import jax.numpy as jnp
from jax import lax
from jax.experimental import pallas as pl
from jax.experimental.pallas import tpu as pltpu
from jax.experimental.pallas import tpu_sc as plsc

D_DIM = 512
VOCAB = 256
SCALE = math.sqrt(float(D_DIM))


def _make_sc_kernel(B: int):
    info = plsc.get_sparse_core_info()
    NC, NS, L = info.num_cores, info.num_subcores, info.num_lanes
    NW = NC * NS
    assert B % NW == 0
    b_per_w = B // NW
    CH = 64
    assert b_per_w % CH == 0
    n_ch = b_per_w // CH
    NBUF = 3
    rows_per_sub = VOCAB // NS

    mesh = plsc.VectorSubcoreMesh(core_axis_name="c", subcore_axis_name="s")

    @functools.partial(
        pl.kernel,
        mesh=mesh,
        out_type=jax.ShapeDtypeStruct((B, D_DIM), jnp.float32),
        scratch_types=[
            pltpu.VMEM((b_per_w,), jnp.int32),
            pltpu.VMEM((rows_per_sub, D_DIM), jnp.float32),
            pltpu.HBM((NC * VOCAB, D_DIM), jnp.float32),
            [pltpu.VMEM((CH, D_DIM), jnp.float32)] * NBUF,
            [pltpu.SemaphoreType.DMA] * NBUF,
            [pltpu.SemaphoreType.DMA] * NBUF,
        ],
    )
    def emb_kernel(x_hbm, table_hbm, out_hbm, idx_v, tslice, stable, bufs,
                   gsems, wsems):
        cid = lax.axis_index("c")
        sid = lax.axis_index("s")
        wid = sid * NC + cid

        row0 = sid * rows_per_sub
        pltpu.sync_copy(table_hbm.at[pl.ds(row0, rows_per_sub)], tslice)
        for r in range(rows_per_sub):
            def scale_body(j, carry, r=r):
                tslice[r, pl.ds(j * L, L)] = tslice[r, pl.ds(j * L, L)] * SCALE
                return carry
            lax.fori_loop(0, D_DIM // L, scale_body, 0)
        pltpu.sync_copy(tslice, stable.at[pl.ds(cid * VOCAB + row0, rows_per_sub)])
        plsc.subcore_barrier()

        base = wid * b_per_w
        pltpu.sync_copy(x_hbm.at[pl.ds(base, b_per_w)], idx_v)
        voff = cid * VOCAB

        def bias_body(j, carry):
            idx_v[pl.ds(j * L, L)] = idx_v[pl.ds(j * L, L)] + voff
            return carry

        lax.fori_loop(0, b_per_w // L, bias_body, 0)

        def gather_start(c, b):
            return pltpu.async_copy(
                stable.at[idx_v.at[pl.ds(c * CH, CH)]], bufs[b], gsems[b]
            )

        gd = [None] * NBUF
        wd = [None] * NBUF
        pending_writes = {}
        gd[0] = gather_start(0, 0)
        if n_ch > 1:
            gd[1] = gather_start(1, 1)
        for c in range(n_ch):
            b = c % NBUF
            gd[b].wait()
            wd[b] = pltpu.async_copy(
                bufs[b], out_hbm.at[pl.ds(base + c * CH, CH)], wsems[b]
            )
            pending_writes[b] = wd[b]
            nxt = c + 2
            if nxt < n_ch:
                bb = nxt % NBUF
                if wd[bb] is not None:
                    wd[bb].wait()
                    pending_writes.pop(bb, None)
                gd[bb] = gather_start(nxt, bb)
        for d in pending_writes.values():
            d.wait()

    return emb_kernel


def kernel(x, table):
    B = x.shape[0] * x.shape[1]
    out = _make_sc_kernel(B)(x.reshape(B), table)
    return out.reshape(x.shape + (D_DIM,))


# speedup vs baseline: 5.0736x; 2.7403x over previous
import functools
import math

import jax
import jax.numpy as jnp
from jax.experimental import pallas as pl
from jax.experimental.pallas import tpu as pltpu

D_DIM = 512
VOCAB = 256
SCALE = math.sqrt(float(D_DIM))
BR = 1024


def _tc_body(x_ref, t_ref, o_ref):
    idx = x_ref[:]
    onehot = (idx[:, None] == jax.lax.broadcasted_iota(jnp.int32, (BR, VOCAB), 1)
              ).astype(jnp.float32)
    o_ref[:] = jnp.dot(onehot, t_ref[:] * SCALE,
                       preferred_element_type=jnp.float32)


@functools.partial(jax.jit, static_argnames=())
def _tc_kernel(xf, table):
    B = xf.shape[0]
    grid = (B // BR,)
    return pl.pallas_call(
        _tc_body,
        grid=grid,
        in_specs=[
            pl.BlockSpec((BR,), lambda i: (i,)),
            pl.BlockSpec((VOCAB, D_DIM), lambda i: (0, 0)),
        ],
        out_specs=pl.BlockSpec((BR, D_DIM), lambda i: (i, 0)),
        out_shape=jax.ShapeDtypeStruct((B, D_DIM), jnp.float32),
    )(xf, table)


def kernel(x, table):
    B = x.shape[0] * x.shape[1]
    out = _tc_kernel(x.reshape(B), table)
    return out.reshape(x.shape + (D_DIM,))
